# Initial kernel scaffold; baseline (speedup 1.0000x reference)
import jax
import jax.numpy as jnp
from jax.experimental import pallas as pl


def kernel(source, rng_seq, rng_idx):
    raise NotImplementedError("write your pallas kernel here")


# speedup vs baseline: 139.8972x; 139.8972x over previous
import functools

import jax
import jax.numpy as jnp
from jax import lax
from jax.experimental import pallas as pl
from jax.experimental.pallas import tpu as pltpu
from jax.experimental.pallas import tpu_sc as plsc

_NC = 2
_NS = 16
_NW = _NC * _NS
_L = 16

_N = 16384 * 128
_PER_W = _N // _NW
_CHUNK = 4096
_NCHUNK = _PER_W // _CHUNK


def _body(src_hbm, rng_hbm, idx_hbm, out_hbm, idx_v, rows_v, src_v, out_v, sem):
    wid = lax.axis_index("s") * _NC + lax.axis_index("c")
    base = wid * _PER_W

    def chunk_body(j, carry):
        off = base + j * _CHUNK
        pltpu.sync_copy(idx_hbm.at[pl.ds(off, _CHUNK)], idx_v)
        gather = pltpu.async_copy(rng_hbm.at[idx_v], rows_v, sem)
        pltpu.sync_copy(src_hbm.at[pl.ds(off, _CHUNK)], src_v)
        gather.wait()

        def vec_body(i, c):
            s = src_v[pl.ds(i * _L, _L)]
            r = rows_v[pl.ds(i * _L, _L)]
            out_v[pl.ds(i * _L, _L)] = jnp.where(s > r, 1.0, 0.0)
            return c

        lax.fori_loop(0, _CHUNK // _L, vec_body, 0, unroll=4)
        pltpu.sync_copy(out_v, out_hbm.at[pl.ds(off, _CHUNK)])
        return carry

    lax.fori_loop(0, _NCHUNK, chunk_body, 0)


@jax.jit
def _run(src_flat, rng_seq, idx_flat):
    mesh = plsc.VectorSubcoreMesh(core_axis_name="c", subcore_axis_name="s")
    return pl.kernel(
        _body,
        out_type=jax.ShapeDtypeStruct((_N,), jnp.float32),
        mesh=mesh,
        scratch_types=[
            pltpu.VMEM((_CHUNK,), jnp.int32),
            pltpu.VMEM((_CHUNK,), jnp.float32),
            pltpu.VMEM((_CHUNK,), jnp.float32),
            pltpu.VMEM((_CHUNK,), jnp.float32),
            pltpu.SemaphoreType.DMA,
        ],
    )(src_flat, rng_seq, idx_flat)


def kernel(source, rng_seq, rng_idx):
    src_flat = source.reshape(-1)
    idx_flat = rng_idx.astype(jnp.int32).reshape(-1)
    out = _run(src_flat, rng_seq, idx_flat)
    return out.reshape(source.shape)


# speedup vs baseline: 182.7142x; 1.3061x over previous
import jax
import jax.numpy as jnp
from jax import lax
from jax.experimental import pallas as pl
from jax.experimental.pallas import tpu as pltpu
from jax.experimental.pallas import tpu_sc as plsc

_NC = 2
_NS = 16
_NW = _NC * _NS
_L = 16

_N = 16384 * 128
_PER_W = _N // _NW
_CHUNK = 4096
_NCHUNK = _PER_W // _CHUNK


def _body(src_hbm, rng_hbm, idx_hbm, out_hbm,
          idx_v0, idx_v1, rows_v0, rows_v1, src_v0, src_v1, out_v0, out_v1,
          idx_sem, gat_sem, src_sem, out_sem):
    wid = lax.axis_index("s") * _NC + lax.axis_index("c")
    base = wid * _PER_W
    idx_b = (idx_v0, idx_v1)
    rows_b = (rows_v0, rows_v1)
    src_b = (src_v0, src_v1)
    out_b = (out_v0, out_v1)

    def span(j):
        return pl.ds(base + j * _CHUNK, _CHUNK)

    def start_idx(j):
        return pltpu.async_copy(idx_hbm.at[span(j)], idx_b[j % 2], idx_sem)

    def start_gather(j):
        return pltpu.async_copy(rng_hbm.at[idx_b[j % 2]], rows_b[j % 2],
                                gat_sem)

    def start_src(j):
        return pltpu.async_copy(src_hbm.at[span(j)], src_b[j % 2], src_sem)

    def start_out(j):
        return pltpu.async_copy(out_b[j % 2], out_hbm.at[span(j)], out_sem)

    def compute(j):
        src_v = src_b[j % 2]
        rows_v = rows_b[j % 2]
        out_v = out_b[j % 2]

        @plsc.parallel_loop(0, _CHUNK // _L, unroll=8)
        def _(i):
            s = src_v[pl.ds(i * _L, _L)]
            r = rows_v[pl.ds(i * _L, _L)]
            out_v[pl.ds(i * _L, _L)] = jnp.where(s > r, 1.0, 0.0)

    idx_d = {0: start_idx(0)}
    idx_d[0].wait()
    gat_d = {0: start_gather(0)}
    src_d = {0: start_src(0)}
    idx_d[1] = start_idx(1)
    out_d = {}
    for j in range(_NCHUNK):
        gat_d[j].wait()
        src_d[j].wait()
        if j + 1 < _NCHUNK:
            idx_d[j + 1].wait()
            gat_d[j + 1] = start_gather(j + 1)
            src_d[j + 1] = start_src(j + 1)
        if j + 2 < _NCHUNK:
            idx_d[j + 2] = start_idx(j + 2)
        if j >= 2:
            out_d[j - 2].wait()
        compute(j)
        out_d[j] = start_out(j)
    out_d[_NCHUNK - 2].wait()
    out_d[_NCHUNK - 1].wait()


@jax.jit
def _run(src_flat, rng_seq, idx_flat):
    mesh = plsc.VectorSubcoreMesh(core_axis_name="c", subcore_axis_name="s")
    return pl.kernel(
        _body,
        out_type=jax.ShapeDtypeStruct((_N,), jnp.float32),
        mesh=mesh,
        scratch_types=[
            pltpu.VMEM((_CHUNK,), jnp.int32),
            pltpu.VMEM((_CHUNK,), jnp.int32),
            pltpu.VMEM((_CHUNK,), jnp.float32),
            pltpu.VMEM((_CHUNK,), jnp.float32),
            pltpu.VMEM((_CHUNK,), jnp.float32),
            pltpu.VMEM((_CHUNK,), jnp.float32),
            pltpu.VMEM((_CHUNK,), jnp.float32),
            pltpu.VMEM((_CHUNK,), jnp.float32),
            pltpu.SemaphoreType.DMA,
            pltpu.SemaphoreType.DMA,
            pltpu.SemaphoreType.DMA,
            pltpu.SemaphoreType.DMA,
        ],
    )(src_flat, rng_seq, idx_flat)


def kernel(source, rng_seq, rng_idx):
    src_flat = source.reshape(-1)
    idx_flat = rng_idx.astype(jnp.int32).reshape(-1)
    out = _run(src_flat, rng_seq, idx_flat)
    return out.reshape(source.shape)


# speedup vs baseline: 188.7440x; 1.0330x over previous
import jax
import jax.numpy as jnp
from jax import lax
from jax.experimental import pallas as pl
from jax.experimental.pallas import tpu as pltpu
from jax.experimental.pallas import tpu_sc as plsc

_NC = 2
_NS = 16
_NW = _NC * _NS
_L = 16

_N = 16384 * 128
_PER_W = _N // _NW
_CHUNK = 8192
_NCHUNK = _PER_W // _CHUNK


def _body(src_hbm, rng_hbm, idx_hbm, out_hbm,
          idx_v0, idx_v1, rows_v0, rows_v1, src_v0, src_v1, out_v0, out_v1,
          idx_sem, gat_sem, src_sem, out_sem):
    wid = lax.axis_index("s") * _NC + lax.axis_index("c")
    base = wid * _PER_W
    idx_b = (idx_v0, idx_v1)
    rows_b = (rows_v0, rows_v1)
    src_b = (src_v0, src_v1)
    out_b = (out_v0, out_v1)

    def span(j):
        return pl.ds(base + j * _CHUNK, _CHUNK)

    def start_idx(j):
        return pltpu.async_copy(idx_hbm.at[span(j)], idx_b[j % 2], idx_sem)

    def start_gather(j):
        return pltpu.async_copy(rng_hbm.at[idx_b[j % 2]], rows_b[j % 2],
                                gat_sem)

    def start_src(j):
        return pltpu.async_copy(src_hbm.at[span(j)], src_b[j % 2], src_sem)

    def start_out(j):
        return pltpu.async_copy(out_b[j % 2], out_hbm.at[span(j)], out_sem)

    def compute(j):
        src_v = src_b[j % 2]
        rows_v = rows_b[j % 2]
        out_v = out_b[j % 2]

        @plsc.parallel_loop(0, _CHUNK // _L, unroll=8)
        def _(i):
            s = src_v[pl.ds(i * _L, _L)]
            r = rows_v[pl.ds(i * _L, _L)]
            out_v[pl.ds(i * _L, _L)] = jnp.where(s > r, 1.0, 0.0)

    idx_d = {0: start_idx(0)}
    idx_d[0].wait()
    gat_d = {0: start_gather(0)}
    src_d = {0: start_src(0)}
    idx_d[1] = start_idx(1)
    out_d = {}
    for j in range(_NCHUNK):
        gat_d[j].wait()
        src_d[j].wait()
        if j + 1 < _NCHUNK:
            idx_d[j + 1].wait()
            gat_d[j + 1] = start_gather(j + 1)
            src_d[j + 1] = start_src(j + 1)
        if j + 2 < _NCHUNK:
            idx_d[j + 2] = start_idx(j + 2)
        if j >= 2:
            out_d[j - 2].wait()
        compute(j)
        out_d[j] = start_out(j)
    out_d[_NCHUNK - 2].wait()
    out_d[_NCHUNK - 1].wait()


@jax.jit
def _run(src_flat, rng_seq, idx_flat):
    mesh = plsc.VectorSubcoreMesh(core_axis_name="c", subcore_axis_name="s")
    return pl.kernel(
        _body,
        out_type=jax.ShapeDtypeStruct((_N,), jnp.float32),
        mesh=mesh,
        scratch_types=[
            pltpu.VMEM((_CHUNK,), jnp.int32),
            pltpu.VMEM((_CHUNK,), jnp.int32),
            pltpu.VMEM((_CHUNK,), jnp.float32),
            pltpu.VMEM((_CHUNK,), jnp.float32),
            pltpu.VMEM((_CHUNK,), jnp.float32),
            pltpu.VMEM((_CHUNK,), jnp.float32),
            pltpu.VMEM((_CHUNK,), jnp.float32),
            pltpu.VMEM((_CHUNK,), jnp.float32),
            pltpu.SemaphoreType.DMA,
            pltpu.SemaphoreType.DMA,
            pltpu.SemaphoreType.DMA,
            pltpu.SemaphoreType.DMA,
        ],
    )(src_flat, rng_seq, idx_flat)


def kernel(source, rng_seq, rng_idx):
    src_flat = source.reshape(-1)
    idx_flat = rng_idx.astype(jnp.int32).reshape(-1)
    out = _run(src_flat, rng_seq, idx_flat)
    return out.reshape(source.shape)
